# baseline (speedup 1.0000x reference)
import functools

import jax
import jax.numpy as jnp
from jax import lax
from jax.experimental import pallas as pl
from jax.experimental.pallas import tpu as pltpu
from jax.experimental.pallas import tpu_sc as plsc

N = 100000
D = 128
S = 512
NW = 32
SEGS_PW = S // NW
CHUNK = 256
NF = D // 16


def _scalar(v):
    return v[0] if getattr(v, "ndim", 0) else v


def _body(x_hbm, ids_hbm, w_hbm, b_hbm, out_hbm,
          xb0, xb1, ib0, ib1, wbuf, bbuf, probuf, acc,
          psem, sx0, sx1, si0, si1):
    wid = lax.axis_index("s") * 2 + lax.axis_index("c")

    pltpu.sync_copy(w_hbm, wbuf)
    pltpu.sync_copy(b_hbm, bbuf)

    zero = jnp.zeros((16,), jnp.float32)
    ninf = jnp.full((16,), -jnp.inf, jnp.float32)
    for s_ in range(SEGS_PW):
        for f_ in range(NF):
            acc[s_, pl.ds(16 * f_, 16)] = zero
            acc[s_, pl.ds(D + 16 * f_, 16)] = ninf

    wvecs = [wbuf[pl.ds(16 * f_, 16)] for f_ in range(NF)]
    bvec = bbuf[...]
    lane = lax.iota(jnp.int32, 16)
    last = jnp.full((16,), 15, jnp.int32)

    seg_base = SEGS_PW * wid

    def search(t):
        def it_body(_, lohi):
            lo, hi = lohi
            step = jnp.maximum((hi - lo + 15) // 16, 1)
            pj = lo + lane * step
            idx = jnp.minimum(pj, N - 1)
            pltpu.async_copy(ids_hbm.at[idx], probuf, psem).wait()
            less = (pj < hi) & (probuf[...] < t)
            c = jnp.sum(less.astype(jnp.int32))
            lo2 = jnp.where(c > 0, lo + (c - 1) * step + 1, lo)
            hi2 = jnp.where(c > 0, jnp.minimum(hi, lo + c * step), lo)
            return lo2, hi2
        lo, _ = lax.fori_loop(0, 6, it_body, (jnp.int32(0), jnp.int32(N)))
        return lo

    start = search(seg_base)
    end = search(seg_base + SEGS_PW)
    nchunks = (end - start + CHUNK - 1) // CHUNK

    def dma_x(c, xb, sem):
        xbase = jnp.minimum(start + c * CHUNK, N - CHUNK)
        return pltpu.make_async_copy(
            x_hbm.at[pl.ds(xbase * D, CHUNK * D)], xb, sem)

    def dma_i(c, ib, sem):
        abase = ((jnp.minimum(start + c * CHUNK, N - CHUNK)) // 8) * 8
        return pltpu.make_async_copy(
            ids_hbm.at[pl.ds(abase, CHUNK + 8)],
            ib.at[pl.ds(0, CHUNK + 8)], sem)

    def process(c, xbuf, idbuf):
        row0 = start + c * CHUNK
        cnt = jnp.minimum(CHUNK, end - row0)
        xbase = jnp.minimum(row0, N - CHUNK)
        xoff = row0 - xbase
        abase = (xbase // 8) * 8
        adelta = xbase - abase
        lim = xoff + cnt

        def one_row(r):
            xv = [xbuf[pl.ds(r * D + 16 * f_, 16)] for f_ in range(NF)]
            p = xv[0] * wvecs[0]
            for f_ in range(1, NF):
                p = p + xv[f_] * wvecs[f_]
            p = plsc.cumsum(p).at[last].get(mode="promise_in_bounds")
            wv = 1.0 / (1.0 + jnp.exp(-(p + bvec)))
            return xv, wv

        def run_cond(st):
            pos = st
            return pos < lim

        def run_body(pos):
            cur = idbuf[pl.ds(pos + adelta, 16)][0]
            sl = cur - seg_base

            def sc_cond(st):
                j, found = st
                return (found == 0) & (j < lim)

            def sc_body(st):
                j, _ = st
                m = idbuf[pl.ds(j + adelta, 16)] != cur
                f = _scalar(plsc.all_reduce_ffs(m))
                return (jnp.where(f < 16, j + f, j + 16).astype(jnp.int32),
                        jnp.where(f < 16, jnp.int32(1), jnp.int32(0)))

            e_j, _ = lax.while_loop(sc_cond, sc_body, (pos, jnp.int32(0)))
            e = jnp.minimum(e_j, lim)

            def tri_body(i, carry):
                sums, maxs = carry
                r = pos + 4 * i
                xv0, wv0 = one_row(r)
                xv1, wv1 = one_row(r + 1)
                xv2, wv2 = one_row(r + 2)
                xv3, wv3 = one_row(r + 3)
                sums = tuple(sums[f_] + ((xv0[f_] * wv0 + xv1[f_] * wv1)
                                         + (xv2[f_] * wv2 + xv3[f_] * wv3))
                             for f_ in range(NF))
                maxs = tuple(jnp.maximum(maxs[f_],
                                         jnp.maximum(
                                             jnp.maximum(xv0[f_], xv1[f_]),
                                             jnp.maximum(xv2[f_], xv3[f_])))
                             for f_ in range(NF))
                return sums, maxs

            def row_body(r, carry):
                sums, maxs = carry
                xv, wv = one_row(r)
                sums = tuple(sums[f_] + xv[f_] * wv for f_ in range(NF))
                maxs = tuple(jnp.maximum(maxs[f_], xv[f_])
                             for f_ in range(NF))
                return sums, maxs

            init = (tuple(zero for _ in range(NF)),
                    tuple(ninf for _ in range(NF)))
            ntri = (e - pos) // 4
            sums, maxs = lax.fori_loop(0, ntri, tri_body, init)
            sums, maxs = lax.fori_loop(
                pos + 4 * ntri, e, row_body, (sums, maxs))

            for f_ in range(NF):
                plsc.addupdate(acc.at[sl, pl.ds(16 * f_, 16)], sums[f_])
                mv = acc[sl, pl.ds(D + 16 * f_, 16)]
                acc[sl, pl.ds(D + 16 * f_, 16)] = jnp.maximum(mv, maxs[f_])
            return e

        lax.while_loop(run_cond, run_body, xoff)

    bufs = [(xb0, ib0, sx0, si0), (xb1, ib1, sx1, si1)]

    @pl.when(nchunks > 0)
    def _():
        dma_x(0, bufs[0][0], bufs[0][2]).start()
        dma_i(0, bufs[0][1], bufs[0][3]).start()

    def gbody(g, _):
        for b_ in range(2):
            c = 2 * g + b_
            xb, ib, sx, si = bufs[b_]
            nxb, nib, nsx, nsi = bufs[1 - b_]

            @pl.when(c < nchunks)
            def _():
                dma_x(c, xb, sx).wait()
                dma_i(c, ib, si).wait()

                @pl.when(c + 1 < nchunks)
                def _():
                    dma_x(c + 1, nxb, nsx).start()
                    dma_i(c + 1, nib, nsi).start()

                process(c, xb, ib)
        return 0

    lax.fori_loop(0, (nchunks + 1) // 2, gbody, 0)
    pltpu.sync_copy(acc, out_hbm.at[pl.ds(SEGS_PW * wid, SEGS_PW)])


@jax.jit
def _run(x, ids, wvec, bvec):
    mesh = plsc.VectorSubcoreMesh(core_axis_name="c", subcore_axis_name="s")
    f = pl.kernel(
        _body,
        out_type=jax.ShapeDtypeStruct((S, 2 * D), jnp.float32),
        mesh=mesh,
        compiler_params=pltpu.CompilerParams(needs_layout_passes=False),
        scratch_types=[
            pltpu.VMEM((CHUNK * D,), jnp.float32),
            pltpu.VMEM((CHUNK * D,), jnp.float32),
            pltpu.VMEM((CHUNK + 24,), jnp.int32),
            pltpu.VMEM((CHUNK + 24,), jnp.int32),
            pltpu.VMEM((D,), jnp.float32),
            pltpu.VMEM((16,), jnp.float32),
            pltpu.VMEM((16,), jnp.int32),
            pltpu.VMEM((SEGS_PW, 2 * D), jnp.float32),
            pltpu.SemaphoreType.DMA,
            pltpu.SemaphoreType.DMA,
            pltpu.SemaphoreType.DMA,
            pltpu.SemaphoreType.DMA,
            pltpu.SemaphoreType.DMA,
        ],
    )
    return f(x, ids, wvec, bvec)


def kernel(x, segment_ids, W, b):
    ids = segment_ids.astype(jnp.int32)
    wvec = W.reshape(D).astype(jnp.float32)
    bvec = jnp.full((16,), b[0], jnp.float32)
    return _run(x.reshape(-1), ids, wvec, bvec)


# speedup vs baseline: 1.3863x; 1.3863x over previous
import functools

import jax
import jax.numpy as jnp
from jax import lax
from jax.experimental import pallas as pl
from jax.experimental.pallas import tpu as pltpu
from jax.experimental.pallas import tpu_sc as plsc

N = 100000
D = 128
S = 512
NW = 32
SEGS_PW = S // NW
CHUNK = 384
NF = D // 16


def _scalar(v):
    return v[0] if getattr(v, "ndim", 0) else v


def _body(x_hbm, ids_hbm, w_hbm, b_hbm, out_hbm,
          xb0, xb1, ib0, ib1, wbuf, bbuf, probuf, acc,
          psem, sx0, sx1, si0, si1):
    wid = lax.axis_index("s") * 2 + lax.axis_index("c")

    pltpu.sync_copy(w_hbm, wbuf)
    pltpu.sync_copy(b_hbm, bbuf)

    zero = jnp.zeros((16,), jnp.float32)
    ninf = jnp.full((16,), -jnp.inf, jnp.float32)
    for s_ in range(SEGS_PW):
        for f_ in range(NF):
            acc[s_, pl.ds(16 * f_, 16)] = zero
            acc[s_, pl.ds(D + 16 * f_, 16)] = ninf

    wvecs = [wbuf[pl.ds(16 * f_, 16)] for f_ in range(NF)]
    bvec = bbuf[...]
    lane = lax.iota(jnp.int32, 16)
    last = jnp.full((16,), 15, jnp.int32)

    seg_base = SEGS_PW * wid

    def search(t):
        def it_body(_, lohi):
            lo, hi = lohi
            step = jnp.maximum((hi - lo + 15) // 16, 1)
            pj = lo + lane * step
            idx = jnp.minimum(pj, N - 1)
            pltpu.async_copy(ids_hbm.at[idx], probuf, psem).wait()
            less = (pj < hi) & (probuf[...] < t)
            c = jnp.sum(less.astype(jnp.int32))
            lo2 = jnp.where(c > 0, lo + (c - 1) * step + 1, lo)
            hi2 = jnp.where(c > 0, jnp.minimum(hi, lo + c * step), lo)
            return lo2, hi2
        lo, _ = lax.fori_loop(0, 6, it_body, (jnp.int32(0), jnp.int32(N)))
        return lo

    start = search(seg_base)
    end = search(seg_base + SEGS_PW)
    nchunks = (end - start + CHUNK - 1) // CHUNK

    def dma_x(c, xb, sem):
        xbase = jnp.minimum(start + c * CHUNK, N - CHUNK)
        return pltpu.make_async_copy(
            x_hbm.at[pl.ds(xbase * D, CHUNK * D)], xb, sem)

    def dma_i(c, ib, sem):
        abase = ((jnp.minimum(start + c * CHUNK, N - CHUNK)) // 8) * 8
        return pltpu.make_async_copy(
            ids_hbm.at[pl.ds(abase, CHUNK + 8)],
            ib.at[pl.ds(0, CHUNK + 8)], sem)

    def process(c, xbuf, idbuf):
        row0 = start + c * CHUNK
        cnt = jnp.minimum(CHUNK, end - row0)
        xbase = jnp.minimum(row0, N - CHUNK)
        xoff = row0 - xbase
        abase = (xbase // 8) * 8
        adelta = xbase - abase
        lim = xoff + cnt

        def one_row(r):
            xv = [xbuf[pl.ds(r * D + 16 * f_, 16)] for f_ in range(NF)]
            p = xv[0] * wvecs[0]
            for f_ in range(1, NF):
                p = p + xv[f_] * wvecs[f_]
            p = plsc.cumsum(p).at[last].get(mode="promise_in_bounds")
            wv = 1.0 / (1.0 + jnp.exp(-(p + bvec)))
            return xv, wv

        def run_cond(st):
            pos = st
            return pos < lim

        def run_body(pos):
            cur = idbuf[pl.ds(pos + adelta, 16)][0]
            sl = cur - seg_base

            def sc_cond(st):
                j, found = st
                return (found == 0) & (j < lim)

            def sc_body(st):
                j, _ = st
                m = idbuf[pl.ds(j + adelta, 16)] != cur
                f = _scalar(plsc.all_reduce_ffs(m))
                return (jnp.where(f < 16, j + f, j + 16).astype(jnp.int32),
                        jnp.where(f < 16, jnp.int32(1), jnp.int32(0)))

            e_j, _ = lax.while_loop(sc_cond, sc_body, (pos, jnp.int32(0)))
            e = jnp.minimum(e_j, lim)

            def tri_body(i, carry):
                sums, maxs = carry
                r = pos + 3 * i
                xv0, wv0 = one_row(r)
                xv1, wv1 = one_row(r + 1)
                xv2, wv2 = one_row(r + 2)
                sums = tuple(sums[f_] + (xv0[f_] * wv0
                                         + (xv1[f_] * wv1 + xv2[f_] * wv2))
                             for f_ in range(NF))
                maxs = tuple(jnp.maximum(maxs[f_],
                                         jnp.maximum(xv0[f_],
                                                     jnp.maximum(xv1[f_],
                                                                 xv2[f_])))
                             for f_ in range(NF))
                return sums, maxs

            def row_body(r, carry):
                sums, maxs = carry
                xv, wv = one_row(r)
                sums = tuple(sums[f_] + xv[f_] * wv for f_ in range(NF))
                maxs = tuple(jnp.maximum(maxs[f_], xv[f_])
                             for f_ in range(NF))
                return sums, maxs

            init = (tuple(zero for _ in range(NF)),
                    tuple(ninf for _ in range(NF)))
            ntri = (e - pos) // 3
            sums, maxs = lax.fori_loop(0, ntri, tri_body, init)
            sums, maxs = lax.fori_loop(
                pos + 3 * ntri, e, row_body, (sums, maxs))

            for f_ in range(NF):
                plsc.addupdate(acc.at[sl, pl.ds(16 * f_, 16)], sums[f_])
                mv = acc[sl, pl.ds(D + 16 * f_, 16)]
                acc[sl, pl.ds(D + 16 * f_, 16)] = jnp.maximum(mv, maxs[f_])
            return e

        lax.while_loop(run_cond, run_body, xoff)

    bufs = [(xb0, ib0, sx0, si0), (xb1, ib1, sx1, si1)]

    @pl.when(nchunks > 0)
    def _():
        dma_x(0, bufs[0][0], bufs[0][2]).start()
        dma_i(0, bufs[0][1], bufs[0][3]).start()

    def gbody(g, _):
        for b_ in range(2):
            c = 2 * g + b_
            xb, ib, sx, si = bufs[b_]
            nxb, nib, nsx, nsi = bufs[1 - b_]

            @pl.when(c < nchunks)
            def _():
                dma_x(c, xb, sx).wait()
                dma_i(c, ib, si).wait()

                @pl.when(c + 1 < nchunks)
                def _():
                    dma_x(c + 1, nxb, nsx).start()
                    dma_i(c + 1, nib, nsi).start()

                process(c, xb, ib)
        return 0

    lax.fori_loop(0, (nchunks + 1) // 2, gbody, 0)
    pltpu.sync_copy(acc, out_hbm.at[pl.ds(SEGS_PW * wid, SEGS_PW)])


@jax.jit
def _run(x, ids, wvec, bvec):
    mesh = plsc.VectorSubcoreMesh(core_axis_name="c", subcore_axis_name="s")
    f = pl.kernel(
        _body,
        out_type=jax.ShapeDtypeStruct((S, 2 * D), jnp.float32),
        mesh=mesh,
        compiler_params=pltpu.CompilerParams(needs_layout_passes=False),
        scratch_types=[
            pltpu.VMEM((CHUNK * D,), jnp.float32),
            pltpu.VMEM((CHUNK * D,), jnp.float32),
            pltpu.VMEM((CHUNK + 24,), jnp.int32),
            pltpu.VMEM((CHUNK + 24,), jnp.int32),
            pltpu.VMEM((D,), jnp.float32),
            pltpu.VMEM((16,), jnp.float32),
            pltpu.VMEM((16,), jnp.int32),
            pltpu.VMEM((SEGS_PW, 2 * D), jnp.float32),
            pltpu.SemaphoreType.DMA,
            pltpu.SemaphoreType.DMA,
            pltpu.SemaphoreType.DMA,
            pltpu.SemaphoreType.DMA,
            pltpu.SemaphoreType.DMA,
        ],
    )
    return f(x, ids, wvec, bvec)


def kernel(x, segment_ids, W, b):
    ids = segment_ids.astype(jnp.int32)
    wvec = W.reshape(D).astype(jnp.float32)
    bvec = jnp.full((16,), b[0], jnp.float32)
    return _run(x.reshape(-1), ids, wvec, bvec)


# speedup vs baseline: 1.4603x; 1.0534x over previous
import functools

import jax
import jax.numpy as jnp
from jax import lax
from jax.experimental import pallas as pl
from jax.experimental.pallas import tpu as pltpu
from jax.experimental.pallas import tpu_sc as plsc

N = 100000
D = 128
S = 512
NW = 32
SEGS_PW = S // NW
CHUNK = 384
NF = D // 16


def _scalar(v):
    return v[0] if getattr(v, "ndim", 0) else v


def _body(x_hbm, ids_hbm, w_hbm, b_hbm, out_hbm,
          xb0, xb1, ib0, ib1, wbuf, bbuf, probuf, acc,
          psem, sx0, sx1, si0, si1):
    wid = lax.axis_index("s") * 2 + lax.axis_index("c")

    pltpu.sync_copy(w_hbm, wbuf)
    pltpu.sync_copy(b_hbm, bbuf)

    zero = jnp.zeros((16,), jnp.float32)
    ninf = jnp.full((16,), -jnp.inf, jnp.float32)
    for s_ in range(SEGS_PW):
        for f_ in range(NF):
            acc[s_, pl.ds(16 * f_, 16)] = zero
            acc[s_, pl.ds(D + 16 * f_, 16)] = ninf

    wvecs = [wbuf[pl.ds(16 * f_, 16)] for f_ in range(NF)]
    bvec = bbuf[...]
    lane = lax.iota(jnp.int32, 16)
    last = jnp.full((16,), 15, jnp.int32)

    seg_base = SEGS_PW * wid

    t1 = seg_base
    t2 = seg_base + SEGS_PW
    lane8 = lane & 7
    is_hi = lane >= 8
    tsel = jnp.where(is_hi, t2, t1)

    def it_body(_, st):
        lo1, hi1, lo2, hi2 = st
        s1 = jnp.maximum((hi1 - lo1 + 7) // 8, 1)
        s2 = jnp.maximum((hi2 - lo2 + 7) // 8, 1)
        pj = jnp.where(is_hi, lo2 + lane8 * s2, lo1 + lane8 * s1)
        valid = pj < jnp.where(is_hi, hi2, hi1)
        idx = jnp.minimum(pj, N - 1)
        pltpu.async_copy(ids_hbm.at[idx], probuf, psem).wait()
        less = (valid & (probuf[...] < tsel)).astype(jnp.int32)
        c1 = jnp.sum(jnp.where(is_hi, 0, less))
        c2 = jnp.sum(jnp.where(is_hi, less, 0))
        nlo1 = jnp.where(c1 > 0, lo1 + (c1 - 1) * s1 + 1, lo1)
        nhi1 = jnp.where(c1 > 0, jnp.minimum(hi1, lo1 + c1 * s1), lo1)
        nlo2 = jnp.where(c2 > 0, lo2 + (c2 - 1) * s2 + 1, lo2)
        nhi2 = jnp.where(c2 > 0, jnp.minimum(hi2, lo2 + c2 * s2), lo2)
        return nlo1, nhi1, nlo2, nhi2

    z, n_ = jnp.int32(0), jnp.int32(N)
    start, _, end, _ = lax.fori_loop(0, 6, it_body, (z, n_, z, n_))
    nchunks = (end - start + CHUNK - 1) // CHUNK

    def dma_x(c, xb, sem):
        xbase = jnp.minimum(start + c * CHUNK, N - CHUNK)
        return pltpu.make_async_copy(
            x_hbm.at[pl.ds(xbase * D, CHUNK * D)], xb, sem)

    def dma_i(c, ib, sem):
        abase = ((jnp.minimum(start + c * CHUNK, N - CHUNK)) // 8) * 8
        return pltpu.make_async_copy(
            ids_hbm.at[pl.ds(abase, CHUNK + 8)],
            ib.at[pl.ds(0, CHUNK + 8)], sem)

    def process(c, xbuf, idbuf):
        row0 = start + c * CHUNK
        cnt = jnp.minimum(CHUNK, end - row0)
        xbase = jnp.minimum(row0, N - CHUNK)
        xoff = row0 - xbase
        abase = (xbase // 8) * 8
        adelta = xbase - abase
        lim = xoff + cnt

        def one_row(r):
            xv = [xbuf[pl.ds(r * D + 16 * f_, 16)] for f_ in range(NF)]
            p = xv[0] * wvecs[0]
            for f_ in range(1, NF):
                p = p + xv[f_] * wvecs[f_]
            p = plsc.cumsum(p).at[last].get(mode="promise_in_bounds")
            wv = 1.0 / (1.0 + jnp.exp(-(p + bvec)))
            return xv, wv

        def run_cond(st):
            pos = st
            return pos < lim

        def run_body(pos):
            cur = idbuf[pl.ds(pos + adelta, 16)][0]
            sl = cur - seg_base

            def sc_cond(st):
                j, found = st
                return (found == 0) & (j < lim)

            def sc_body(st):
                j, _ = st
                m = idbuf[pl.ds(j + adelta, 16)] != cur
                f = _scalar(plsc.all_reduce_ffs(m))
                return (jnp.where(f < 16, j + f, j + 16).astype(jnp.int32),
                        jnp.where(f < 16, jnp.int32(1), jnp.int32(0)))

            e_j, _ = lax.while_loop(sc_cond, sc_body, (pos, jnp.int32(0)))
            e = jnp.minimum(e_j, lim)

            def tri_body(i, carry):
                sums, maxs = carry
                r = pos + 3 * i
                xv0, wv0 = one_row(r)
                xv1, wv1 = one_row(r + 1)
                xv2, wv2 = one_row(r + 2)
                sums = tuple(sums[f_] + (xv0[f_] * wv0
                                         + (xv1[f_] * wv1 + xv2[f_] * wv2))
                             for f_ in range(NF))
                maxs = tuple(jnp.maximum(maxs[f_],
                                         jnp.maximum(xv0[f_],
                                                     jnp.maximum(xv1[f_],
                                                                 xv2[f_])))
                             for f_ in range(NF))
                return sums, maxs

            def row_body(r, carry):
                sums, maxs = carry
                xv, wv = one_row(r)
                sums = tuple(sums[f_] + xv[f_] * wv for f_ in range(NF))
                maxs = tuple(jnp.maximum(maxs[f_], xv[f_])
                             for f_ in range(NF))
                return sums, maxs

            init = (tuple(zero for _ in range(NF)),
                    tuple(ninf for _ in range(NF)))
            ntri = (e - pos) // 3
            sums, maxs = lax.fori_loop(0, ntri, tri_body, init)
            sums, maxs = lax.fori_loop(
                pos + 3 * ntri, e, row_body, (sums, maxs))

            for f_ in range(NF):
                plsc.addupdate(acc.at[sl, pl.ds(16 * f_, 16)], sums[f_])
                mv = acc[sl, pl.ds(D + 16 * f_, 16)]
                acc[sl, pl.ds(D + 16 * f_, 16)] = jnp.maximum(mv, maxs[f_])
            return e

        lax.while_loop(run_cond, run_body, xoff)

    bufs = [(xb0, ib0, sx0, si0), (xb1, ib1, sx1, si1)]

    @pl.when(nchunks > 0)
    def _():
        dma_x(0, bufs[0][0], bufs[0][2]).start()
        dma_i(0, bufs[0][1], bufs[0][3]).start()

    def gbody(g, _):
        for b_ in range(2):
            c = 2 * g + b_
            xb, ib, sx, si = bufs[b_]
            nxb, nib, nsx, nsi = bufs[1 - b_]

            @pl.when(c < nchunks)
            def _():
                dma_x(c, xb, sx).wait()
                dma_i(c, ib, si).wait()

                @pl.when(c + 1 < nchunks)
                def _():
                    dma_x(c + 1, nxb, nsx).start()
                    dma_i(c + 1, nib, nsi).start()

                process(c, xb, ib)
        return 0

    lax.fori_loop(0, (nchunks + 1) // 2, gbody, 0)
    pltpu.sync_copy(acc, out_hbm.at[pl.ds(SEGS_PW * wid, SEGS_PW)])


@jax.jit
def _run(x, ids, wvec, bvec):
    mesh = plsc.VectorSubcoreMesh(core_axis_name="c", subcore_axis_name="s")
    f = pl.kernel(
        _body,
        out_type=jax.ShapeDtypeStruct((S, 2 * D), jnp.float32),
        mesh=mesh,
        compiler_params=pltpu.CompilerParams(needs_layout_passes=False),
        scratch_types=[
            pltpu.VMEM((CHUNK * D,), jnp.float32),
            pltpu.VMEM((CHUNK * D,), jnp.float32),
            pltpu.VMEM((CHUNK + 24,), jnp.int32),
            pltpu.VMEM((CHUNK + 24,), jnp.int32),
            pltpu.VMEM((D,), jnp.float32),
            pltpu.VMEM((16,), jnp.float32),
            pltpu.VMEM((16,), jnp.int32),
            pltpu.VMEM((SEGS_PW, 2 * D), jnp.float32),
            pltpu.SemaphoreType.DMA,
            pltpu.SemaphoreType.DMA,
            pltpu.SemaphoreType.DMA,
            pltpu.SemaphoreType.DMA,
            pltpu.SemaphoreType.DMA,
        ],
    )
    return f(x, ids, wvec, bvec)


def kernel(x, segment_ids, W, b):
    ids = segment_ids.astype(jnp.int32)
    wvec = W.reshape(D).astype(jnp.float32)
    bvec = jnp.full((16,), b[0], jnp.float32)
    return _run(x.reshape(-1), ids, wvec, bvec)
